# baseline (speedup 1.0000x reference)
import functools

import numpy as np

import jax
import jax.numpy as jnp
from jax.experimental import pallas as pl
from jax.experimental.pallas import tpu as pltpu

_F = 12
_M = 256


def _round_up(a, m):
    return (a + m - 1) // m * m


def _fused_kernel(x_ref, t1_ref, b1_ref, t2_ref, f1_ref, f1b_ref,
                  f2_ref, f2b_ref, b2_ref, o_ref):
    x2 = x_ref[...]
    bf = jnp.bfloat16
    h1 = jnp.maximum(
        jnp.dot(x2, t1_ref[...], preferred_element_type=jnp.float32)
        + b1_ref[...], 0.0).astype(bf)
    y2 = jnp.maximum(
        jnp.dot(h1, t2_ref[...], preferred_element_type=jnp.float32)
        + b2_ref[0], 0.0).astype(bf)
    h = jnp.maximum(
        jnp.dot(y2, f1_ref[...], preferred_element_type=jnp.float32)
        + f1b_ref[...], 0.0).astype(bf)
    o_ref[...] = (jnp.dot(h, f2_ref[...], preferred_element_type=jnp.float32)
                  + f2b_ref[...])


def kernel(x, conv1_w, conv1_b, conv2_w, conv2_b, fc1_w, fc1_b, fc2_w, fc2_b):
    N, C, S, F = x.shape
    assert C == 1 and F == _F
    n_ch = conv1_w.shape[0]
    hid = fc1_w.shape[0]
    bf = jnp.bfloat16

    npad = _round_up(max(N, 1), _M)
    nblocks = npad // _M

    xs = x.reshape(N, S * F).astype(bf)
    if npad != N:
        xs = jnp.pad(xs, ((0, npad - N), (0, 0)))

    bk = np.stack([np.kron(np.eye(S, k=1 - j, dtype=np.float32),
                           np.eye(F, dtype=np.float32)) for j in range(3)])
    bk = jnp.asarray(bk)

    w1m = conv1_w.reshape(n_ch, 3).astype(jnp.float32)
    w2m = conv2_w.reshape(n_ch, 3).astype(jnp.float32)
    t1 = jnp.einsum('cj,jpq->pcq', w1m, bk).reshape(S * F, n_ch * S * F)
    t2 = jnp.einsum('cj,jpq->cpq', w2m, bk).reshape(n_ch * S * F, S * F)
    b1big = jnp.repeat(conv1_b.astype(jnp.float32), S * F).reshape(1, -1)

    eye = jnp.eye(S, dtype=jnp.float32)
    f1 = jnp.kron(eye, fc1_w.T.astype(jnp.float32))
    f1b = jnp.tile(fc1_b.astype(jnp.float32), S).reshape(1, S * hid)

    blend = np.zeros((2 * S, S + 1), np.float32)
    blend[0, 0] = 1.0
    for s in range(1, S):
        blend[2 * s - 1, s] = 0.5
        blend[2 * s, s] = 0.5
    blend[2 * S - 1, S] = 1.0
    blend = jnp.asarray(blend)
    f2 = jnp.kron(eye, fc2_w.T.astype(jnp.float32)) @ blend
    f2b = (jnp.tile(fc2_b.astype(jnp.float32), S) @ blend).reshape(1, S + 1)

    smem = pl.BlockSpec(memory_space=pltpu.MemorySpace.SMEM)
    full = lambda r, c: pl.BlockSpec((r, c), lambda b: (0, 0))

    out = pl.pallas_call(
        _fused_kernel,
        out_shape=jax.ShapeDtypeStruct((npad, S + 1), jnp.float32),
        grid=(nblocks,),
        in_specs=[
            pl.BlockSpec((_M, S * F), lambda b: (b, 0)),
            full(S * F, n_ch * S * F),
            full(1, n_ch * S * F),
            full(n_ch * S * F, S * F),
            full(S * F, S * hid),
            full(1, S * hid),
            full(S * hid, S + 1),
            full(1, S + 1),
            smem,
        ],
        out_specs=pl.BlockSpec((_M, S + 1), lambda b: (b, 0)),
        compiler_params=pltpu.CompilerParams(
            dimension_semantics=("parallel",),
            vmem_limit_bytes=96 * 1024 * 1024),
    )(xs, t1.astype(bf), b1big, t2.astype(bf), f1.astype(bf), f1b,
      f2.astype(bf), f2b, conv2_b.astype(jnp.float32))

    return out[:N]
